# Initial kernel scaffold; baseline (speedup 1.0000x reference)
import jax
import jax.numpy as jnp
from jax.experimental import pallas as pl


def kernel(xyz, feats, cost_volume, flow, params):
    raise NotImplementedError("write your pallas kernel here")


# speedup vs baseline: 3.8693x; 3.8693x over previous
import functools

import jax
import jax.numpy as jnp
from jax.experimental import pallas as pl
from jax.experimental.pallas import tpu as pltpu
from jax.experimental.pallas import tpu_sc as plsc

LEAKY = 0.1
EPS = 1e-5
KNBR = 9

BLK_KNN = 256
BLK_PC = 256
BLK_SMALL = 512


def _knn_body(xyzt_ref, xyz_ref, ext_ref, idx_ref, aux_ref):
    f32 = jnp.float32
    xb = xyzt_ref[0]
    xa = xyz_ref[0]
    ext = ext_ref[0]
    blk = xb.shape[0]
    mm = jnp.dot(xb, xa, preferred_element_type=f32)
    ssrc = jnp.sum(xb * xb, axis=1, keepdims=True)
    sdst = jnp.sum(xa * xa, axis=0, keepdims=True)
    d = (-2.0 * mm + ssrc) + sdst
    col = jax.lax.broadcasted_iota(jnp.int32, d.shape, 1)
    centerext = jnp.concatenate([xb, jnp.zeros((blk, 5), f32)], axis=1)
    pieces = []
    for k in range(KNBR):
        m = jnp.min(d, axis=1, keepdims=True)
        cand = jnp.where(d == m, col, jnp.int32(2**30))
        j = jnp.min(cand, axis=1, keepdims=True)
        idx_ref[0, :, k : k + 1] = j
        oh = col == j
        d = jnp.where(oh, jnp.inf, d)
        gx = jnp.dot(oh.astype(f32), ext,
                     precision=jax.lax.Precision.HIGHEST,
                     preferred_element_type=f32)
        pieces.append(gx - centerext)
    pieces.append(jnp.zeros((blk, 128 - 8 * KNBR), f32))
    aux_ref[0] = jnp.concatenate(pieces, axis=1)


def _knn(xyz_t, xyz, ext):
    b, n, _ = xyz_t.shape
    return pl.pallas_call(
        _knn_body,
        grid=(b, n // BLK_KNN),
        in_specs=[
            pl.BlockSpec((1, BLK_KNN, 3), lambda bi, i: (bi, i, 0)),
            pl.BlockSpec((1, 3, n), lambda bi, i: (bi, 0, 0)),
            pl.BlockSpec((1, n, 8), lambda bi, i: (bi, 0, 0)),
        ],
        out_specs=[
            pl.BlockSpec((1, BLK_KNN, KNBR), lambda bi, i: (bi, i, 0)),
            pl.BlockSpec((1, BLK_KNN, 128), lambda bi, i: (bi, i, 0)),
        ],
        out_shape=[
            jax.ShapeDtypeStruct((b, n, KNBR), jnp.int32),
            jax.ShapeDtypeStruct((b, n, 128), jnp.float32),
        ],
    )(xyz_t, xyz, ext)


def _sc_gather(p_flat, idx_flat):
    m = idx_flat.shape[1]
    win = 128
    cdim = p_flat.shape[1]
    mesh = plsc.VectorSubcoreMesh(core_axis_name="core", subcore_axis_name="subcore")

    @pl.kernel(
        out_type=jax.ShapeDtypeStruct((m, cdim), p_flat.dtype),
        mesh=mesh,
    )
    def kern(x_hbm, i_hbm, o_hbm):
        def body(i_vmem, o_vmem):
            pltpu.sync_copy(x_hbm.at[i_vmem.at[0]], o_vmem)

        pltpu.emit_pipeline(
            body,
            grid=(m // win,),
            in_specs=[pl.BlockSpec((1, win), index_map=lambda i: (0, i))],
            out_specs=[pl.BlockSpec((win, cdim), index_map=lambda i: (i, 0))],
            core_axis_name=("core", "subcore"),
            dimension_semantics=(pltpu.PARALLEL,),
        )(i_hbm, o_hbm)

    return kern(p_flat, idx_flat)


def _pc_body(g_ref, aux_ref, w1_ref, b1_ref, w2_ref, b2_ref, w3_ref, b3_ref,
             l2a_ref, l2f_ref, lb_ref, out_ref, sums_ref):
    aux = aux_ref[0]
    gs = []
    a8s = []
    ws = []
    for k in range(KNBR):
        a8 = aux[:, 8 * k : 8 * k + 8]
        a8s.append(a8)
        gs.append(g_ref[0, k])
        h = jnp.maximum(jnp.dot(a8, w1_ref[...], preferred_element_type=jnp.float32) + b1_ref[...], 0.0)
        h = jnp.maximum(jnp.dot(h, w2_ref[...], preferred_element_type=jnp.float32) + b2_ref[...], 0.0)
        h = jnp.maximum(jnp.dot(h, w3_ref[...], preferred_element_type=jnp.float32) + b3_ref[...], 0.0)
        ws.append(h)
    acc = jnp.zeros((out_ref.shape[1], out_ref.shape[2]), jnp.float32)
    for w in range(16):
        mw = gs[0] * ws[0][:, w : w + 1]
        aw = a8s[0] * ws[0][:, w : w + 1]
        for k in range(1, KNBR):
            mw = mw + gs[k] * ws[k][:, w : w + 1]
            aw = aw + a8s[k] * ws[k][:, w : w + 1]
        acc = acc + jnp.dot(mw, l2f_ref[w], preferred_element_type=jnp.float32)
        acc = acc + jnp.dot(aw, l2a_ref[w], preferred_element_type=jnp.float32)
    acc = acc + lb_ref[...]
    out_ref[0] = acc

    @pl.when(jnp.logical_and(pl.program_id(0) == 0, pl.program_id(1) == 0))
    def _():
        sums_ref[...] = jnp.zeros_like(sums_ref)

    sums_ref[0:1, :] = sums_ref[0:1, :] + jnp.sum(acc, axis=0, keepdims=True)
    sums_ref[1:2, :] = sums_ref[1:2, :] + jnp.sum(acc * acc, axis=0, keepdims=True)


def _pointconv(g, aux, wn, l2a, l2f, lb):
    b, _, n, _ = g.shape
    (w1, b1), (w2, b2), (w3, b3) = wn
    out, sums = pl.pallas_call(
        _pc_body,
        grid=(b, n // BLK_PC),
        in_specs=[
            pl.BlockSpec((1, KNBR, BLK_PC, 128), lambda bi, i: (bi, 0, i, 0)),
            pl.BlockSpec((1, BLK_PC, 128), lambda bi, i: (bi, i, 0)),
            pl.BlockSpec((8, 8), lambda bi, i: (0, 0)),
            pl.BlockSpec((1, 8), lambda bi, i: (0, 0)),
            pl.BlockSpec((8, 8), lambda bi, i: (0, 0)),
            pl.BlockSpec((1, 8), lambda bi, i: (0, 0)),
            pl.BlockSpec((8, 16), lambda bi, i: (0, 0)),
            pl.BlockSpec((1, 16), lambda bi, i: (0, 0)),
            pl.BlockSpec((16, 8, 128), lambda bi, i: (0, 0, 0)),
            pl.BlockSpec((16, 128, 128), lambda bi, i: (0, 0, 0)),
            pl.BlockSpec((1, 128), lambda bi, i: (0, 0)),
        ],
        out_specs=[
            pl.BlockSpec((1, BLK_PC, 128), lambda bi, i: (bi, i, 0)),
            pl.BlockSpec((8, 128), lambda bi, i: (0, 0)),
        ],
        out_shape=[
            jax.ShapeDtypeStruct((b, n, 128), jnp.float32),
            jax.ShapeDtypeStruct((8, 128), jnp.float32),
        ],
    )(g, aux, w1, b1, w2, b2, w3, b3, l2a, l2f, lb)
    return out, sums


def _norm_body(cnt, x_ref, sums_ref, gam_ref, bet_ref, y_ref):
    mean = sums_ref[0:1, :] / cnt
    var = sums_ref[1:2, :] / cnt - mean * mean
    inv = jax.lax.rsqrt(var + EPS)
    x = x_ref[0]
    y = (x - mean) * (inv * gam_ref[...]) + bet_ref[...]
    y_ref[0] = jnp.where(y >= 0, y, LEAKY * y)


def _norm_apply(x, sums, gamma, beta):
    b, n, _ = x.shape
    cnt = float(b * n)
    return pl.pallas_call(
        functools.partial(_norm_body, cnt),
        grid=(b, n // BLK_SMALL),
        in_specs=[
            pl.BlockSpec((1, BLK_SMALL, 128), lambda bi, i: (bi, i, 0)),
            pl.BlockSpec((8, 128), lambda bi, i: (0, 0)),
            pl.BlockSpec((1, 128), lambda bi, i: (0, 0)),
            pl.BlockSpec((1, 128), lambda bi, i: (0, 0)),
        ],
        out_specs=pl.BlockSpec((1, BLK_SMALL, 128), lambda bi, i: (bi, i, 0)),
        out_shape=jax.ShapeDtypeStruct((b, n, 128), jnp.float32),
    )(x, sums, gamma, beta)


def _final_body(cnt, x_ref, sums_ref, gam_ref, bet_ref, m1w_ref, m1b_ref,
                m2w_ref, m2b_ref, fcw_ref, fcb_ref, np_ref, fl_ref):
    mean = sums_ref[0:1, :] / cnt
    var = sums_ref[1:2, :] / cnt - mean * mean
    inv = jax.lax.rsqrt(var + EPS)
    x = x_ref[0]
    y = (x - mean) * (inv * gam_ref[...]) + bet_ref[...]
    y = jnp.where(y >= 0, y, LEAKY * y)
    h = jnp.dot(y, m1w_ref[...], preferred_element_type=jnp.float32) + m1b_ref[...]
    h = jnp.where(h >= 0, h, LEAKY * h)
    h = jnp.dot(h, m2w_ref[...], preferred_element_type=jnp.float32) + m2b_ref[...]
    h = jnp.where(h >= 0, h, LEAKY * h)
    np_ref[0] = h
    fl = jnp.dot(h, fcw_ref[...], preferred_element_type=jnp.float32) + fcb_ref[...]
    fl_ref[0] = jnp.clip(fl, -200.0, 200.0)


def _final(x, sums, gamma, beta, m1w, m1b, m2w, m2b, fcw, fcb):
    b, n, _ = x.shape
    cnt = float(b * n)
    return pl.pallas_call(
        functools.partial(_final_body, cnt),
        grid=(b, n // BLK_SMALL),
        in_specs=[
            pl.BlockSpec((1, BLK_SMALL, 128), lambda bi, i: (bi, i, 0)),
            pl.BlockSpec((8, 128), lambda bi, i: (0, 0)),
            pl.BlockSpec((1, 128), lambda bi, i: (0, 0)),
            pl.BlockSpec((1, 128), lambda bi, i: (0, 0)),
            pl.BlockSpec((128, 128), lambda bi, i: (0, 0)),
            pl.BlockSpec((1, 128), lambda bi, i: (0, 0)),
            pl.BlockSpec((128, 64), lambda bi, i: (0, 0)),
            pl.BlockSpec((1, 64), lambda bi, i: (0, 0)),
            pl.BlockSpec((64, 8), lambda bi, i: (0, 0)),
            pl.BlockSpec((1, 8), lambda bi, i: (0, 0)),
        ],
        out_specs=[
            pl.BlockSpec((1, BLK_SMALL, 64), lambda bi, i: (bi, i, 0)),
            pl.BlockSpec((1, BLK_SMALL, 8), lambda bi, i: (bi, i, 0)),
        ],
        out_shape=[
            jax.ShapeDtypeStruct((b, n, 64), jnp.float32),
            jax.ShapeDtypeStruct((b, n, 8), jnp.float32),
        ],
    )(x, sums, gamma, beta, m1w, m1b, m2w, m2b, fcw, fcb)


def _prep_pc_weights(pc, creal, has_flow):
    (w1, b1), (w2, b2), (w3, b3) = pc["wn"]
    w1p = jnp.zeros((8, 8), jnp.float32).at[0:3, :].set(w1.T)
    wn = ((w1p, b1[None, :]), (w2.T, b2[None, :]), (w3.T, b3[None, :]))
    l2 = pc["lin_W"].reshape(128, creal, 16).transpose(2, 1, 0)
    l2a = jnp.zeros((16, 8, 128), jnp.float32)
    l2a = l2a.at[:, 0:3, :].set(l2[:, 0:3, :])
    if has_flow:
        l2a = l2a.at[:, 3:6, :].set(l2[:, creal - 3 :, :])
    l2f = l2[:, 3:131, :]
    return wn, l2a, l2f, pc["lin_b"][None, :]


def kernel(xyz, feats, cost_volume, flow, params):
    b, _, n = xyz.shape
    f32 = jnp.float32
    xyz_t = xyz.transpose(0, 2, 1)
    flow_t = flow.transpose(0, 2, 1)
    ext = jnp.concatenate([xyz_t, flow_t, jnp.zeros((b, n, 2), f32)], axis=-1)

    idx, aux = _knn(xyz_t, xyz, ext)
    offs = (jnp.arange(b, dtype=jnp.int32) * n)[:, None, None]
    idx_flat = (idx.transpose(0, 2, 1) + offs).reshape(1, b * KNBR * n)

    src1 = jnp.concatenate([feats, cost_volume], axis=1).transpose(0, 2, 1)
    g1 = _sc_gather(src1.reshape(b * n, 128), idx_flat).reshape(b, KNBR, n, 128)
    pc1 = params["pointconvs"][0]
    wn1, l2a1, l2f1, lb1 = _prep_pc_weights(pc1, 134, True)
    out1, sums1 = _pointconv(g1, aux, wn1, l2a1, l2f1, lb1)
    p2 = _norm_apply(out1, sums1, pc1["gamma"][None, :], pc1["beta"][None, :])

    g2 = _sc_gather(p2.reshape(b * n, 128), idx_flat).reshape(b, KNBR, n, 128)
    pc2 = params["pointconvs"][1]
    wn2, l2a2, l2f2, lb2 = _prep_pc_weights(pc2, 131, False)
    out2, sums2 = _pointconv(g2, aux, wn2, l2a2, l2f2, lb2)

    (m1w, m1b), (m2w, m2b) = params["mlps"]
    fcw = jnp.zeros((64, 8), f32).at[:, 0:3].set(params["fc_W"].T)
    fcb = jnp.zeros((1, 8), f32).at[0, 0:3].set(params["fc_b"])
    np_out, fl_out = _final(
        out2, sums2, pc2["gamma"][None, :], pc2["beta"][None, :],
        m1w.T, m1b[None, :], m2w.T, m2b[None, :], fcw, fcb)

    return np_out.transpose(0, 2, 1), fl_out[..., 0:3].transpose(0, 2, 1)


# speedup vs baseline: 10.6304x; 2.7474x over previous
import functools

import jax
import jax.numpy as jnp
from jax.experimental import pallas as pl
from jax.experimental.pallas import tpu as pltpu
from jax.experimental.pallas import tpu_sc as plsc

LEAKY = 0.1
EPS = 1e-5
KNBR = 9

BLK_KNN = 256
BLK_PC = 256
BLK_SMALL = 512


def _knn_body(xyzt_ref, xyz_ref, idx_ref):
    f32 = jnp.float32
    xb = xyzt_ref[0]
    xa = xyz_ref[0]
    mm = jnp.dot(xb, xa, preferred_element_type=f32)
    ssrc = jnp.sum(xb * xb, axis=1, keepdims=True)
    sdst = jnp.sum(xa * xa, axis=0, keepdims=True)
    d = (-2.0 * mm + ssrc) + sdst
    col = jax.lax.broadcasted_iota(jnp.int32, d.shape, 1)
    for k in range(KNBR):
        m = jnp.min(d, axis=1, keepdims=True)
        cand = jnp.where(d == m, col, jnp.int32(2**30))
        j = jnp.min(cand, axis=1, keepdims=True)
        idx_ref[0, :, k : k + 1] = j
        d = jnp.where(col == j, jnp.inf, d)


def _knn(xyz_t, xyz):
    b, n, _ = xyz_t.shape
    return pl.pallas_call(
        _knn_body,
        grid=(b, n // BLK_KNN),
        in_specs=[
            pl.BlockSpec((1, BLK_KNN, 3), lambda bi, i: (bi, i, 0)),
            pl.BlockSpec((1, 3, n), lambda bi, i: (bi, 0, 0)),
        ],
        out_specs=pl.BlockSpec((1, BLK_KNN, KNBR), lambda bi, i: (bi, i, 0)),
        out_shape=jax.ShapeDtypeStruct((b, n, KNBR), jnp.int32),
    )(xyz_t, xyz)


def _sc_gather(p_flat, idx_flat):
    m = idx_flat.shape[1]
    win = 128
    cdim = p_flat.shape[1]
    mesh = plsc.VectorSubcoreMesh(core_axis_name="core", subcore_axis_name="subcore")

    @pl.kernel(
        out_type=jax.ShapeDtypeStruct((m, cdim), p_flat.dtype),
        mesh=mesh,
    )
    def kern(x_hbm, i_hbm, o_hbm):
        def body(i_vmem, o_vmem):
            pltpu.sync_copy(x_hbm.at[i_vmem.at[0]], o_vmem)

        pltpu.emit_pipeline(
            body,
            grid=(m // win,),
            in_specs=[pl.BlockSpec((1, win), index_map=lambda i: (0, i))],
            out_specs=[pl.BlockSpec((win, cdim), index_map=lambda i: (i, 0))],
            core_axis_name=("core", "subcore"),
            dimension_semantics=(pltpu.PARALLEL,),
        )(i_hbm, o_hbm)

    return kern(p_flat, idx_flat)


def _pc_body(g_ref, ge_ref, cen_ref, w1_ref, b1_ref, w2_ref, b2_ref, w3_ref,
             b3_ref, l2a_ref, l2f_ref, lb_ref, out_ref, sums_ref):
    cen = cen_ref[0]
    gs = []
    a8s = []
    ws = []
    for k in range(KNBR):
        a8 = ge_ref[0, k][:, 0:8] - cen
        a8s.append(a8)
        gs.append(g_ref[0, k])
        h = jnp.maximum(jnp.dot(a8, w1_ref[...], preferred_element_type=jnp.float32) + b1_ref[...], 0.0)
        h = jnp.maximum(jnp.dot(h, w2_ref[...], preferred_element_type=jnp.float32) + b2_ref[...], 0.0)
        h = jnp.maximum(jnp.dot(h, w3_ref[...], preferred_element_type=jnp.float32) + b3_ref[...], 0.0)
        ws.append(h)
    mws = []
    aws = []
    for w in range(16):
        mw = gs[0] * ws[0][:, w : w + 1]
        aw = a8s[0] * ws[0][:, w : w + 1]
        for k in range(1, KNBR):
            mw = mw + gs[k] * ws[k][:, w : w + 1]
            aw = aw + a8s[k] * ws[k][:, w : w + 1]
        mws.append(mw)
        aws.append(aw)
    mcat = jnp.concatenate(mws, axis=1)
    acat = jnp.concatenate(aws, axis=1)
    acc = jnp.dot(mcat, l2f_ref[...], preferred_element_type=jnp.float32)
    acc = acc + jnp.dot(acat, l2a_ref[...], preferred_element_type=jnp.float32)
    acc = acc + lb_ref[...]
    out_ref[0] = acc

    @pl.when(jnp.logical_and(pl.program_id(0) == 0, pl.program_id(1) == 0))
    def _():
        sums_ref[...] = jnp.zeros_like(sums_ref)

    sums_ref[0:1, :] = sums_ref[0:1, :] + jnp.sum(acc, axis=0, keepdims=True)
    sums_ref[1:2, :] = sums_ref[1:2, :] + jnp.sum(acc * acc, axis=0, keepdims=True)


def _pointconv(g, gext, cen, wn, l2a, l2f, lb):
    b, _, n, _ = g.shape
    (w1, b1), (w2, b2), (w3, b3) = wn
    out, sums = pl.pallas_call(
        _pc_body,
        grid=(b, n // BLK_PC),
        in_specs=[
            pl.BlockSpec((1, KNBR, BLK_PC, 128), lambda bi, i: (bi, 0, i, 0)),
            pl.BlockSpec((1, KNBR, BLK_PC, 128), lambda bi, i: (bi, 0, i, 0)),
            pl.BlockSpec((1, BLK_PC, 8), lambda bi, i: (bi, i, 0)),
            pl.BlockSpec((8, 8), lambda bi, i: (0, 0)),
            pl.BlockSpec((1, 8), lambda bi, i: (0, 0)),
            pl.BlockSpec((8, 8), lambda bi, i: (0, 0)),
            pl.BlockSpec((1, 8), lambda bi, i: (0, 0)),
            pl.BlockSpec((8, 16), lambda bi, i: (0, 0)),
            pl.BlockSpec((1, 16), lambda bi, i: (0, 0)),
            pl.BlockSpec((128, 128), lambda bi, i: (0, 0)),
            pl.BlockSpec((16 * 128, 128), lambda bi, i: (0, 0)),
            pl.BlockSpec((1, 128), lambda bi, i: (0, 0)),
        ],
        out_specs=[
            pl.BlockSpec((1, BLK_PC, 128), lambda bi, i: (bi, i, 0)),
            pl.BlockSpec((8, 128), lambda bi, i: (0, 0)),
        ],
        out_shape=[
            jax.ShapeDtypeStruct((b, n, 128), jnp.float32),
            jax.ShapeDtypeStruct((8, 128), jnp.float32),
        ],
    )(g, gext, cen, w1, b1, w2, b2, w3, b3, l2a, l2f, lb)
    return out, sums


def _norm_body(cnt, x_ref, sums_ref, gam_ref, bet_ref, y_ref):
    mean = sums_ref[0:1, :] / cnt
    var = sums_ref[1:2, :] / cnt - mean * mean
    inv = jax.lax.rsqrt(var + EPS)
    x = x_ref[0]
    y = (x - mean) * (inv * gam_ref[...]) + bet_ref[...]
    y_ref[0] = jnp.where(y >= 0, y, LEAKY * y)


def _norm_apply(x, sums, gamma, beta):
    b, n, _ = x.shape
    cnt = float(b * n)
    return pl.pallas_call(
        functools.partial(_norm_body, cnt),
        grid=(b, n // BLK_SMALL),
        in_specs=[
            pl.BlockSpec((1, BLK_SMALL, 128), lambda bi, i: (bi, i, 0)),
            pl.BlockSpec((8, 128), lambda bi, i: (0, 0)),
            pl.BlockSpec((1, 128), lambda bi, i: (0, 0)),
            pl.BlockSpec((1, 128), lambda bi, i: (0, 0)),
        ],
        out_specs=pl.BlockSpec((1, BLK_SMALL, 128), lambda bi, i: (bi, i, 0)),
        out_shape=jax.ShapeDtypeStruct((b, n, 128), jnp.float32),
    )(x, sums, gamma, beta)


def _final_body(cnt, x_ref, sums_ref, gam_ref, bet_ref, m1w_ref, m1b_ref,
                m2w_ref, m2b_ref, fcw_ref, fcb_ref, np_ref, fl_ref):
    mean = sums_ref[0:1, :] / cnt
    var = sums_ref[1:2, :] / cnt - mean * mean
    inv = jax.lax.rsqrt(var + EPS)
    x = x_ref[0]
    y = (x - mean) * (inv * gam_ref[...]) + bet_ref[...]
    y = jnp.where(y >= 0, y, LEAKY * y)
    h = jnp.dot(y, m1w_ref[...], preferred_element_type=jnp.float32) + m1b_ref[...]
    h = jnp.where(h >= 0, h, LEAKY * h)
    h = jnp.dot(h, m2w_ref[...], preferred_element_type=jnp.float32) + m2b_ref[...]
    h = jnp.where(h >= 0, h, LEAKY * h)
    np_ref[0] = h
    fl = jnp.dot(h, fcw_ref[...], preferred_element_type=jnp.float32) + fcb_ref[...]
    fl_ref[0] = jnp.clip(fl, -200.0, 200.0)


def _final(x, sums, gamma, beta, m1w, m1b, m2w, m2b, fcw, fcb):
    b, n, _ = x.shape
    cnt = float(b * n)
    return pl.pallas_call(
        functools.partial(_final_body, cnt),
        grid=(b, n // BLK_SMALL),
        in_specs=[
            pl.BlockSpec((1, BLK_SMALL, 128), lambda bi, i: (bi, i, 0)),
            pl.BlockSpec((8, 128), lambda bi, i: (0, 0)),
            pl.BlockSpec((1, 128), lambda bi, i: (0, 0)),
            pl.BlockSpec((1, 128), lambda bi, i: (0, 0)),
            pl.BlockSpec((128, 128), lambda bi, i: (0, 0)),
            pl.BlockSpec((1, 128), lambda bi, i: (0, 0)),
            pl.BlockSpec((128, 64), lambda bi, i: (0, 0)),
            pl.BlockSpec((1, 64), lambda bi, i: (0, 0)),
            pl.BlockSpec((64, 8), lambda bi, i: (0, 0)),
            pl.BlockSpec((1, 8), lambda bi, i: (0, 0)),
        ],
        out_specs=[
            pl.BlockSpec((1, BLK_SMALL, 64), lambda bi, i: (bi, i, 0)),
            pl.BlockSpec((1, BLK_SMALL, 8), lambda bi, i: (bi, i, 0)),
        ],
        out_shape=[
            jax.ShapeDtypeStruct((b, n, 64), jnp.float32),
            jax.ShapeDtypeStruct((b, n, 8), jnp.float32),
        ],
    )(x, sums, gamma, beta, m1w, m1b, m2w, m2b, fcw, fcb)


def _prep_pc_weights(pc, creal, has_flow):
    (w1, b1), (w2, b2), (w3, b3) = pc["wn"]
    w1p = jnp.zeros((8, 8), jnp.float32).at[0:3, :].set(w1.T)
    wn = ((w1p, b1[None, :]), (w2.T, b2[None, :]), (w3.T, b3[None, :]))
    l2 = pc["lin_W"].reshape(128, creal, 16).transpose(2, 1, 0)
    l2a = jnp.zeros((16, 8, 128), jnp.float32)
    l2a = l2a.at[:, 0:3, :].set(l2[:, 0:3, :])
    if has_flow:
        l2a = l2a.at[:, 3:6, :].set(l2[:, creal - 3 :, :])
    l2f = l2[:, 3:131, :]
    return wn, l2a.reshape(16 * 8, 128), l2f.reshape(16 * 128, 128), pc["lin_b"][None, :]


def kernel(xyz, feats, cost_volume, flow, params):
    b, _, n = xyz.shape
    f32 = jnp.float32
    xyz_t = xyz.transpose(0, 2, 1)
    flow_t = flow.transpose(0, 2, 1)

    idx = _knn(xyz_t, xyz)
    offs = (jnp.arange(b, dtype=jnp.int32) * n)[:, None, None]
    idx_flat = (idx.transpose(0, 2, 1) + offs).reshape(1, b * KNBR * n)

    ext = jnp.concatenate(
        [xyz_t, flow_t, jnp.zeros((b, n, 122), f32)], axis=-1)
    gext = _sc_gather(ext.reshape(b * n, 128), idx_flat).reshape(b, KNBR, n, 128)
    cen = jnp.concatenate([xyz_t, jnp.zeros((b, n, 5), f32)], axis=-1)

    src1 = jnp.concatenate([feats, cost_volume], axis=1).transpose(0, 2, 1)
    g1 = _sc_gather(src1.reshape(b * n, 128), idx_flat).reshape(b, KNBR, n, 128)
    pc1 = params["pointconvs"][0]
    wn1, l2a1, l2f1, lb1 = _prep_pc_weights(pc1, 134, True)
    out1, sums1 = _pointconv(g1, gext, cen, wn1, l2a1, l2f1, lb1)
    p2 = _norm_apply(out1, sums1, pc1["gamma"][None, :], pc1["beta"][None, :])

    g2 = _sc_gather(p2.reshape(b * n, 128), idx_flat).reshape(b, KNBR, n, 128)
    pc2 = params["pointconvs"][1]
    wn2, l2a2, l2f2, lb2 = _prep_pc_weights(pc2, 131, False)
    out2, sums2 = _pointconv(g2, gext, cen, wn2, l2a2, l2f2, lb2)

    (m1w, m1b), (m2w, m2b) = params["mlps"]
    fcw = jnp.zeros((64, 8), f32).at[:, 0:3].set(params["fc_W"].T)
    fcb = jnp.zeros((1, 8), f32).at[0, 0:3].set(params["fc_b"])
    np_out, fl_out = _final(
        out2, sums2, pc2["gamma"][None, :], pc2["beta"][None, :],
        m1w.T, m1b[None, :], m2w.T, m2b[None, :], fcw, fcb)

    return np_out.transpose(0, 2, 1), fl_out[..., 0:3].transpose(0, 2, 1)


# speedup vs baseline: 16.7085x; 1.5718x over previous
import functools

import jax
import jax.numpy as jnp
from jax.experimental import pallas as pl
from jax.experimental.pallas import tpu as pltpu
from jax.experimental.pallas import tpu_sc as plsc

LEAKY = 0.1
EPS = 1e-5
KNBR = 9

BLK_KNN = 256
BLK_PC = 256
BLK_SMALL = 512


def _knn_body(xyzt_ref, xyz_ref, idx_ref):
    f32 = jnp.float32
    xb = xyzt_ref[0]
    xa = xyz_ref[0]
    mm = jnp.dot(xb, xa, preferred_element_type=f32)
    ssrc = jnp.sum(xb * xb, axis=1, keepdims=True)
    sdst = jnp.sum(xa * xa, axis=0, keepdims=True)
    d = (-2.0 * mm + ssrc) + sdst
    colf = jax.lax.broadcasted_iota(jnp.int32, d.shape, 1).astype(f32)
    for k in range(KNBR):
        m = jnp.min(d, axis=1, keepdims=True)
        cand = jnp.where(d == m, colf, f32(2.0**30))
        jf = jnp.min(cand, axis=1, keepdims=True)
        idx_ref[0, :, k : k + 1] = jf.astype(jnp.int32)
        d = jnp.where(colf == jf, jnp.inf, d)


def _knn(xyz_t, xyz):
    b, n, _ = xyz_t.shape
    return pl.pallas_call(
        _knn_body,
        grid=(b, n // BLK_KNN),
        in_specs=[
            pl.BlockSpec((1, BLK_KNN, 3), lambda bi, i: (bi, i, 0)),
            pl.BlockSpec((1, 3, n), lambda bi, i: (bi, 0, 0)),
        ],
        out_specs=pl.BlockSpec((1, BLK_KNN, KNBR), lambda bi, i: (bi, i, 0)),
        out_shape=jax.ShapeDtypeStruct((b, n, KNBR), jnp.int32),
    )(xyz_t, xyz)


def _sc_gather(p_flat, idx_flat):
    m = idx_flat.shape[1]
    win = 128
    cdim = p_flat.shape[1]
    mesh = plsc.VectorSubcoreMesh(core_axis_name="core", subcore_axis_name="subcore")

    @pl.kernel(
        out_type=jax.ShapeDtypeStruct((m, cdim), p_flat.dtype),
        mesh=mesh,
    )
    def kern(x_hbm, i_hbm, o_hbm):
        def body(i_vmem, o_vmem):
            pltpu.sync_copy(x_hbm.at[i_vmem.at[0]], o_vmem)

        pltpu.emit_pipeline(
            body,
            grid=(m // win,),
            in_specs=[pl.BlockSpec((1, win), index_map=lambda i: (0, i))],
            out_specs=[pl.BlockSpec((win, cdim), index_map=lambda i: (i, 0))],
            core_axis_name=("core", "subcore"),
            dimension_semantics=(pltpu.PARALLEL,),
        )(i_hbm, o_hbm)

    return kern(p_flat, idx_flat)


def _pc_body(g_ref, ge_ref, cen_ref, w1_ref, b1_ref, w2_ref, b2_ref, w3_ref,
             b3_ref, l2a_ref, l2f_ref, lb_ref, out_ref, sums_ref):
    cen = cen_ref[0]
    gs = []
    a8s = []
    ws = []
    for k in range(KNBR):
        a8 = ge_ref[0, k][:, 0:8] - cen
        a8s.append(a8)
        gs.append(g_ref[0, k])
        h = jnp.maximum(jnp.dot(a8, w1_ref[...], preferred_element_type=jnp.float32) + b1_ref[...], 0.0)
        h = jnp.maximum(jnp.dot(h, w2_ref[...], preferred_element_type=jnp.float32) + b2_ref[...], 0.0)
        h = jnp.maximum(jnp.dot(h, w3_ref[...], preferred_element_type=jnp.float32) + b3_ref[...], 0.0)
        ws.append(h)
    bf = jnp.bfloat16
    f32 = jnp.float32
    gs = [g.astype(bf).astype(f32) for g in gs]
    ws = [w.astype(bf).astype(f32) for w in ws]
    mws = []
    for w in range(16):
        mw = gs[0] * ws[0][:, w : w + 1]
        for k in range(1, KNBR):
            mw = mw + gs[k] * ws[k][:, w : w + 1]
        mws.append(mw)
    mcat = jnp.concatenate(mws, axis=1)
    sl = jax.lax.broadcasted_iota(jnp.int32, (8, 128), 1)
    sr = jax.lax.broadcasted_iota(jnp.int32, (8, 128), 0)
    ssel = (sl // 16 == sr).astype(f32)
    tl = jax.lax.broadcasted_iota(jnp.int32, (16, 128), 1)
    tr = jax.lax.broadcasted_iota(jnp.int32, (16, 128), 0)
    tsel = (tl % 16 == tr).astype(f32)
    bcat = None
    for k in range(KNBR):
        aexp = jnp.dot(a8s[k], ssel, preferred_element_type=f32)
        wexp = jnp.dot(ws[k], tsel, preferred_element_type=f32)
        bcat = aexp * wexp if bcat is None else bcat + aexp * wexp
    acc = jnp.dot(mcat, l2f_ref[...], preferred_element_type=jnp.float32)
    acc = acc + jnp.dot(bcat, l2a_ref[...], preferred_element_type=jnp.float32)
    acc = acc + lb_ref[...]
    out_ref[0] = acc

    @pl.when(jnp.logical_and(pl.program_id(0) == 0, pl.program_id(1) == 0))
    def _():
        sums_ref[...] = jnp.zeros_like(sums_ref)

    sums_ref[0:1, :] = sums_ref[0:1, :] + jnp.sum(acc, axis=0, keepdims=True)
    sums_ref[1:2, :] = sums_ref[1:2, :] + jnp.sum(acc * acc, axis=0, keepdims=True)


def _pointconv(g, gext, cen, wn, l2a, l2f, lb):
    b, _, n, _ = g.shape
    (w1, b1), (w2, b2), (w3, b3) = wn
    out, sums = pl.pallas_call(
        _pc_body,
        grid=(b, n // BLK_PC),
        in_specs=[
            pl.BlockSpec((1, KNBR, BLK_PC, 128), lambda bi, i: (bi, 0, i, 0)),
            pl.BlockSpec((1, KNBR, BLK_PC, 128), lambda bi, i: (bi, 0, i, 0)),
            pl.BlockSpec((1, BLK_PC, 8), lambda bi, i: (bi, i, 0)),
            pl.BlockSpec((8, 8), lambda bi, i: (0, 0)),
            pl.BlockSpec((1, 8), lambda bi, i: (0, 0)),
            pl.BlockSpec((8, 8), lambda bi, i: (0, 0)),
            pl.BlockSpec((1, 8), lambda bi, i: (0, 0)),
            pl.BlockSpec((8, 16), lambda bi, i: (0, 0)),
            pl.BlockSpec((1, 16), lambda bi, i: (0, 0)),
            pl.BlockSpec((128, 128), lambda bi, i: (0, 0)),
            pl.BlockSpec((16 * 128, 128), lambda bi, i: (0, 0)),
            pl.BlockSpec((1, 128), lambda bi, i: (0, 0)),
        ],
        out_specs=[
            pl.BlockSpec((1, BLK_PC, 128), lambda bi, i: (bi, i, 0)),
            pl.BlockSpec((8, 128), lambda bi, i: (0, 0)),
        ],
        out_shape=[
            jax.ShapeDtypeStruct((b, n, 128), jnp.float32),
            jax.ShapeDtypeStruct((8, 128), jnp.float32),
        ],
    )(g, gext, cen, w1, b1, w2, b2, w3, b3, l2a, l2f, lb)
    return out, sums


def _norm_body(cnt, x_ref, sums0_ref, sums1_ref, gam_ref, bet_ref, y_ref):
    sums = sums0_ref[...] + sums1_ref[...]
    mean = sums[0:1, :] / cnt
    var = sums[1:2, :] / cnt - mean * mean
    inv = jax.lax.rsqrt(var + EPS)
    x = x_ref[0]
    y = (x - mean) * (inv * gam_ref[...]) + bet_ref[...]
    y_ref[0] = jnp.where(y >= 0, y, LEAKY * y)


def _norm_apply(x, sums0, sums1, cnt, gamma, beta):
    b, n, _ = x.shape
    return pl.pallas_call(
        functools.partial(_norm_body, cnt),
        grid=(b, n // BLK_SMALL),
        in_specs=[
            pl.BlockSpec((1, BLK_SMALL, 128), lambda bi, i: (bi, i, 0)),
            pl.BlockSpec((8, 128), lambda bi, i: (0, 0)),
            pl.BlockSpec((8, 128), lambda bi, i: (0, 0)),
            pl.BlockSpec((1, 128), lambda bi, i: (0, 0)),
            pl.BlockSpec((1, 128), lambda bi, i: (0, 0)),
        ],
        out_specs=pl.BlockSpec((1, BLK_SMALL, 128), lambda bi, i: (bi, i, 0)),
        out_shape=jax.ShapeDtypeStruct((b, n, 128), jnp.float32),
    )(x, sums0, sums1, gamma, beta)


def _final_body(cnt, x_ref, sums0_ref, sums1_ref, gam_ref, bet_ref, m1w_ref,
                m1b_ref, m2w_ref, m2b_ref, fcw_ref, fcb_ref, np_ref, fl_ref):
    sums = sums0_ref[...] + sums1_ref[...]
    mean = sums[0:1, :] / cnt
    var = sums[1:2, :] / cnt - mean * mean
    inv = jax.lax.rsqrt(var + EPS)
    x = x_ref[0]
    y = (x - mean) * (inv * gam_ref[...]) + bet_ref[...]
    y = jnp.where(y >= 0, y, LEAKY * y)
    h = jnp.dot(y, m1w_ref[...], preferred_element_type=jnp.float32) + m1b_ref[...]
    h = jnp.where(h >= 0, h, LEAKY * h)
    h = jnp.dot(h, m2w_ref[...], preferred_element_type=jnp.float32) + m2b_ref[...]
    h = jnp.where(h >= 0, h, LEAKY * h)
    np_ref[0] = h
    fl = jnp.dot(h, fcw_ref[...], preferred_element_type=jnp.float32) + fcb_ref[...]
    fl_ref[0] = jnp.clip(fl, -200.0, 200.0)


def _final(x, sums0, sums1, cnt, gamma, beta, m1w, m1b, m2w, m2b, fcw, fcb):
    b, n, _ = x.shape
    return pl.pallas_call(
        functools.partial(_final_body, cnt),
        grid=(b, n // BLK_SMALL),
        in_specs=[
            pl.BlockSpec((1, BLK_SMALL, 128), lambda bi, i: (bi, i, 0)),
            pl.BlockSpec((8, 128), lambda bi, i: (0, 0)),
            pl.BlockSpec((8, 128), lambda bi, i: (0, 0)),
            pl.BlockSpec((1, 128), lambda bi, i: (0, 0)),
            pl.BlockSpec((1, 128), lambda bi, i: (0, 0)),
            pl.BlockSpec((128, 128), lambda bi, i: (0, 0)),
            pl.BlockSpec((1, 128), lambda bi, i: (0, 0)),
            pl.BlockSpec((128, 64), lambda bi, i: (0, 0)),
            pl.BlockSpec((1, 64), lambda bi, i: (0, 0)),
            pl.BlockSpec((64, 8), lambda bi, i: (0, 0)),
            pl.BlockSpec((1, 8), lambda bi, i: (0, 0)),
        ],
        out_specs=[
            pl.BlockSpec((1, BLK_SMALL, 64), lambda bi, i: (bi, i, 0)),
            pl.BlockSpec((1, BLK_SMALL, 8), lambda bi, i: (bi, i, 0)),
        ],
        out_shape=[
            jax.ShapeDtypeStruct((b, n, 64), jnp.float32),
            jax.ShapeDtypeStruct((b, n, 8), jnp.float32),
        ],
    )(x, sums0, sums1, gamma, beta, m1w, m1b, m2w, m2b, fcw, fcb)


def _prep_pc_weights(pc, creal, has_flow):
    (w1, b1), (w2, b2), (w3, b3) = pc["wn"]
    w1p = jnp.zeros((8, 8), jnp.float32).at[0:3, :].set(w1.T)
    wn = ((w1p, b1[None, :]), (w2.T, b2[None, :]), (w3.T, b3[None, :]))
    l2 = pc["lin_W"].reshape(128, creal, 16).transpose(2, 1, 0)
    l2a = jnp.zeros((16, 8, 128), jnp.float32)
    l2a = l2a.at[:, 0:3, :].set(l2[:, 0:3, :])
    if has_flow:
        l2a = l2a.at[:, 3:6, :].set(l2[:, creal - 3 :, :])
    l2a = l2a.transpose(1, 0, 2)
    l2f = l2[:, 3:131, :]
    return wn, l2a.reshape(8 * 16, 128), l2f.reshape(16 * 128, 128), pc["lin_b"][None, :]


def kernel(xyz, feats, cost_volume, flow, params):
    b, _, n = xyz.shape
    f32 = jnp.float32
    cnt = float(b * n)
    xyz_t = xyz.transpose(0, 2, 1)
    flow_t = flow.transpose(0, 2, 1)

    ext = jnp.concatenate(
        [xyz_t, flow_t, jnp.zeros((b, n, 122), f32)], axis=-1)
    cen = jnp.concatenate([xyz_t, jnp.zeros((b, n, 5), f32)], axis=-1)
    src1 = jnp.concatenate([feats, cost_volume], axis=1).transpose(0, 2, 1)

    pc1 = params["pointconvs"][0]
    pc2 = params["pointconvs"][1]
    wn1, l2a1, l2f1, lb1 = _prep_pc_weights(pc1, 134, True)
    wn2, l2a2, l2f2, lb2 = _prep_pc_weights(pc2, 131, False)

    idxf, gext, g1 = [], [], []
    for i in range(b):
        idx_i = _knn(xyz_t[i : i + 1], xyz[i : i + 1])
        idxf_i = idx_i.transpose(0, 2, 1).reshape(1, KNBR * n)
        idxf.append(idxf_i)
        gext.append(_sc_gather(ext[i], idxf_i).reshape(1, KNBR, n, 128))
        g1.append(_sc_gather(src1[i], idxf_i).reshape(1, KNBR, n, 128))

    out1, sums1 = [], []
    for i in range(b):
        o, s = _pointconv(g1[i], gext[i], cen[i : i + 1], wn1, l2a1, l2f1, lb1)
        out1.append(o)
        sums1.append(s)

    p2, g2 = [], []
    for i in range(b):
        p2_i = _norm_apply(out1[i], sums1[0], sums1[1], cnt,
                           pc1["gamma"][None, :], pc1["beta"][None, :])
        p2.append(p2_i)
        g2.append(_sc_gather(p2_i[0], idxf[i]).reshape(1, KNBR, n, 128))

    out2, sums2 = [], []
    for i in range(b):
        o, s = _pointconv(g2[i], gext[i], cen[i : i + 1], wn2, l2a2, l2f2, lb2)
        out2.append(o)
        sums2.append(s)

    (m1w, m1b), (m2w, m2b) = params["mlps"]
    fcw = jnp.zeros((64, 8), f32).at[:, 0:3].set(params["fc_W"].T)
    fcb = jnp.zeros((1, 8), f32).at[0, 0:3].set(params["fc_b"])
    np_out, fl_out = [], []
    for i in range(b):
        npo, flo = _final(
            out2[i], sums2[0], sums2[1], cnt,
            pc2["gamma"][None, :], pc2["beta"][None, :],
            m1w.T, m1b[None, :], m2w.T, m2b[None, :], fcw, fcb)
        np_out.append(npo)
        fl_out.append(flo)

    np_out = jnp.concatenate(np_out, axis=0)
    fl_out = jnp.concatenate(fl_out, axis=0)
    return np_out.transpose(0, 2, 1), fl_out[..., 0:3].transpose(0, 2, 1)
